# baseline (speedup 1.0000x reference)
import functools

import jax
import jax.numpy as jnp
from jax import lax
from jax.experimental import pallas as pl
from jax.experimental.pallas import tpu as pltpu, tpu_sc as plsc

N = 10000
E = 320000
D = 128
H = 64
C = 4

NC = 2
NS = 16
NW = NC * NS
NP = 10112
RPS = NP // NS

CH = 128
NCH = 80
NBUF = 4
EPW = NCH * CH
EP = NW * EPW
DW = 16

_mesh = plsc.VectorSubcoreMesh(core_axis_name="c", subcore_axis_name="s")
_sc_params = pltpu.CompilerParams(use_tc_tiling_on_sc=False)


@functools.partial(
    pl.kernel,
    out_type=jax.ShapeDtypeStruct((NC, NP, DW), jnp.float32),
    mesh=_mesh,
    compiler_params=_sc_params,
    scratch_types=[
        pltpu.VMEM((NCH, CH), jnp.int32),
        pltpu.VMEM((CH, DW), jnp.float32),
        pltpu.VMEM_SHARED((NP, DW), jnp.float32),
        pltpu.SemaphoreType.DMA,
    ],
)
def _deg_kernel(dst_hbm, ones_hbm, zero_hbm, out_hbm, dst_v, ones_v, deg_sh,
                sem):
  c = lax.axis_index("c")
  s = lax.axis_index("s")
  wid = s * NC + c
  row0 = pl.multiple_of(s * RPS, 8)

  pltpu.sync_copy(zero_hbm.at[pl.ds(row0, RPS)], deg_sh.at[pl.ds(row0, RPS)])
  pltpu.sync_copy(dst_hbm.at[wid], dst_v)
  pltpu.sync_copy(ones_hbm, ones_v)
  plsc.subcore_barrier()

  GRP = 16

  @pl.loop(0, NCH // GRP)
  def _(g):
    descs = []
    for b in range(GRP):
      descs.append(pltpu.async_copy(
          ones_v, deg_sh.at[dst_v.at[g * GRP + b]], sem, add=True))
    for d in descs:
      d.wait()

  plsc.subcore_barrier()
  pltpu.sync_copy(deg_sh.at[pl.ds(row0, RPS)],
                  out_hbm.at[c, pl.ds(row0, RPS)])


@functools.partial(
    pl.kernel,
    out_type=jax.ShapeDtypeStruct((NC, NP, H), jnp.float32),
    mesh=_mesh,
    compiler_params=_sc_params,
    scratch_types=[
        pltpu.VMEM((NCH, CH), jnp.int32),
        pltpu.VMEM((NCH, CH), jnp.int32),
        [pltpu.VMEM((CH, H), jnp.float32) for _ in range(NBUF)],
        [pltpu.SemaphoreType.DMA for _ in range(NBUF)],
        [pltpu.SemaphoreType.DMA for _ in range(NBUF)],
        pltpu.VMEM_SHARED((NP, H), jnp.float32),
    ],
)
def _msg_kernel(y_hbm, src_hbm, dst_hbm, zero_hbm, out_hbm,
                src_v, dst_v, bufs, gsems, ssems, z_sh):
  c = lax.axis_index("c")
  s = lax.axis_index("s")
  wid = s * NC + c
  row0 = pl.multiple_of(s * RPS, 8)

  pltpu.sync_copy(zero_hbm.at[pl.ds(row0, RPS)], z_sh.at[pl.ds(row0, RPS)])
  pltpu.sync_copy(src_hbm.at[wid], src_v)
  pltpu.sync_copy(dst_hbm.at[wid], dst_v)
  plsc.subcore_barrier()

  for b in range(NBUF):
    pltpu.async_copy(y_hbm.at[src_v.at[b]], bufs[b], gsems[b])

  @pl.loop(0, NCH // NBUF)
  def _(g):
    descs = []
    for b in range(NBUF):
      j = g * NBUF + b
      pltpu.make_async_copy(y_hbm.at[src_v.at[0]], bufs[b], gsems[b]).wait()
      descs.append(pltpu.async_copy(
          bufs[b], z_sh.at[dst_v.at[j]], ssems[b], add=True))
    for b in range(NBUF):
      descs[b].wait()
      nxt = jnp.minimum(g * NBUF + b + NBUF, NCH - 1)
      pltpu.async_copy(y_hbm.at[src_v.at[nxt]], bufs[b], gsems[b])

  for b in range(NBUF):
    pltpu.make_async_copy(y_hbm.at[src_v.at[0]], bufs[b], gsems[b]).wait()

  plsc.subcore_barrier()
  pltpu.sync_copy(z_sh.at[pl.ds(row0, RPS)],
                  out_hbm.at[c, pl.ds(row0, RPS)])


def _xw_body(x_ref, w_ref, xw_ref):
  xw_ref[...] = jnp.dot(x_ref[...], w_ref[...],
                        preferred_element_type=jnp.float32)


def _xw_call(x_p, w):
  return pl.pallas_call(
      _xw_body,
      out_shape=jax.ShapeDtypeStruct((NP, H), jnp.float32),
  )(x_p, w)


def _scale_body(xw_ref, deg_ref, y_ref, dinv_ref):
  deg = deg_ref[0, :, 0] + deg_ref[1, :, 0] + 1.0
  dinv = lax.rsqrt(deg)
  y_ref[...] = xw_ref[...] * dinv[:, None]
  dinv_ref[...] = dinv[:, None]


def _scale_call(xw, deg_parts):
  return pl.pallas_call(
      _scale_body,
      out_shape=(jax.ShapeDtypeStruct((NP, H), jnp.float32),
                 jax.ShapeDtypeStruct((NP, 1), jnp.float32)),
  )(xw, deg_parts)


def _head_body(z_ref, y_ref, dinv_ref, bg_ref, w1_ref, b1_ref, w2_ref, b2_ref,
               w3_ref, b3_ref, o_ref):
  z = z_ref[0] + z_ref[1] + y_ref[...]
  h = jax.nn.relu(z * dinv_ref[...] + bg_ref[...])
  h = jax.nn.relu(
      jnp.dot(h, w1_ref[...], preferred_element_type=jnp.float32) + b1_ref[...])
  h = jax.nn.relu(
      jnp.dot(h, w2_ref[...], preferred_element_type=jnp.float32) + b2_ref[...])
  h = jnp.dot(h, w3_ref[...], preferred_element_type=jnp.float32) + b3_ref[...]
  m = jnp.max(h, axis=1, keepdims=True)
  lse = jnp.log(jnp.sum(jnp.exp(h - m), axis=1, keepdims=True))
  o_ref[...] = h - m - lse


def _head_call(z_parts, y, dinv, bg, w1, b1, w2, b2, w3, b3):
  return pl.pallas_call(
      _head_body,
      out_shape=jax.ShapeDtypeStruct((NP, C), jnp.float32),
  )(z_parts, y, dinv, bg, w1, b1, w2, b2, w3, b3)


def kernel(x, edge_index, W_gcn, b_gcn, W1, b1, W2, b2, W3, b3):
  src = edge_index[0]
  dst = edge_index[1]

  pad = jnp.full((EP - E,), N, dtype=jnp.int32)
  src_p = jnp.concatenate([src, pad]).reshape(NW, NCH, CH)
  dst_p = jnp.concatenate([dst, pad]).reshape(NW, NCH, CH)
  x_p = jnp.pad(x, ((0, NP - N), (0, 0)))
  zeros = jnp.zeros((NP, H), jnp.float32)
  ones_rows = jnp.ones((CH, DW), jnp.float32)
  zeros_rows = jnp.zeros((NP, DW), jnp.float32)

  deg_parts = _deg_kernel(dst_p, ones_rows, zeros_rows)
  xw = _xw_call(x_p, W_gcn)
  y, dinv = _scale_call(xw, deg_parts)
  z_parts = _msg_kernel(y, src_p, dst_p, zeros)
  out = _head_call(z_parts, y, dinv,
                   b_gcn.reshape(1, H), W1, b1.reshape(1, 32),
                   W2, b2.reshape(1, 16), W3, b3.reshape(1, C))
  return out[:N]


# speedup vs baseline: 1.8569x; 1.8569x over previous
import functools

import jax
import jax.numpy as jnp
from jax import lax
from jax.experimental import pallas as pl
from jax.experimental.pallas import tpu as pltpu, tpu_sc as plsc

N = 10000
E = 320000
D = 128
H = 64
C = 4

NC = 2
NS = 16
NW = NC * NS
NP = 10112
RPS = NP // NS

EPW = E // NW
CH = 128
NCH = EPW // CH
CHT = EPW - NCH * CH
DW = 16

_mesh = plsc.VectorSubcoreMesh(core_axis_name="c", subcore_axis_name="s")
_sc_params = pltpu.CompilerParams(use_tc_tiling_on_sc=False)


def _zero_rows(buf, rows, width):
  zeros16 = jnp.zeros((16,), jnp.float32)

  @pl.loop(0, rows)
  def _(i):
    for k in range(width // 16):
      buf[i, pl.ds(k * 16, 16)] = zeros16


def _init_shared(zbuf, shared, row0):
  for t in range(RPS // CH):
    pltpu.sync_copy(zbuf, shared.at[pl.ds(row0 + t * CH, CH)])
  rem = RPS - (RPS // CH) * CH
  pltpu.sync_copy(zbuf.at[pl.ds(0, rem)],
                  shared.at[pl.ds(row0 + (RPS // CH) * CH, rem)])


@functools.partial(
    pl.kernel,
    out_type=jax.ShapeDtypeStruct((NC, NP, DW), jnp.float32),
    mesh=_mesh,
    compiler_params=_sc_params,
    scratch_types=[
        pltpu.VMEM((EPW,), jnp.int32),
        pltpu.VMEM((CH, DW), jnp.float32),
        pltpu.VMEM((CH, DW), jnp.float32),
        pltpu.VMEM_SHARED((NP, DW), jnp.float32),
    ],
)
def _deg_kernel(ei_hbm, out_hbm, dst_v, ones_v, zbuf, deg_sh):
  c = lax.axis_index("c")
  s = lax.axis_index("s")
  wid = s * NC + c
  row0 = pl.multiple_of(s * RPS, 8)
  base = pl.multiple_of(wid * EPW, 8)

  ones16 = jnp.ones((16,), jnp.float32)

  @pl.loop(0, CH)
  def _(i):
    ones_v[i, pl.ds(0, DW)] = ones16

  _zero_rows(zbuf, CH, DW)
  _init_shared(zbuf, deg_sh, row0)
  pltpu.sync_copy(ei_hbm.at[1, pl.ds(base, EPW)], dst_v)

  plsc.subcore_barrier()

  @pl.loop(0, NCH)
  def _(j):
    pltpu.sync_copy(ones_v, deg_sh.at[dst_v.at[pl.ds(j * CH, CH)]], add=True)

  pltpu.sync_copy(ones_v.at[pl.ds(0, CHT)],
                  deg_sh.at[dst_v.at[pl.ds(NCH * CH, CHT)]], add=True)

  plsc.subcore_barrier()
  pltpu.sync_copy(deg_sh.at[pl.ds(row0, RPS)],
                  out_hbm.at[c, pl.ds(row0, RPS)])


@functools.partial(
    pl.kernel,
    out_type=jax.ShapeDtypeStruct((NC, NP, H), jnp.float32),
    mesh=_mesh,
    compiler_params=_sc_params,
    scratch_types=[
        pltpu.VMEM((EPW,), jnp.int32),
        pltpu.VMEM((EPW,), jnp.int32),
        pltpu.VMEM((CH, H), jnp.float32),
        pltpu.VMEM((CH, H), jnp.float32),
        pltpu.VMEM_SHARED((NP, H), jnp.float32),
        pltpu.SemaphoreType.DMA,
    ],
)
def _msg_kernel(y_hbm, ei_hbm, out_hbm, src_v, dst_v, rows_v, zbuf, z_sh, sem):
  c = lax.axis_index("c")
  s = lax.axis_index("s")
  wid = s * NC + c
  row0 = pl.multiple_of(s * RPS, 8)
  base = pl.multiple_of(wid * EPW, 8)

  _zero_rows(zbuf, CH, H)
  _init_shared(zbuf, z_sh, row0)
  pltpu.sync_copy(ei_hbm.at[0, pl.ds(base, EPW)], src_v)
  pltpu.sync_copy(ei_hbm.at[1, pl.ds(base, EPW)], dst_v)
  plsc.subcore_barrier()

  @pl.loop(0, NCH)
  def _(j):
    pltpu.async_copy(
        y_hbm.at[src_v.at[pl.ds(j * CH, CH)]], rows_v, sem).wait()
    pltpu.sync_copy(rows_v, z_sh.at[dst_v.at[pl.ds(j * CH, CH)]], add=True)

  pltpu.async_copy(
      y_hbm.at[src_v.at[pl.ds(NCH * CH, CHT)]],
      rows_v.at[pl.ds(0, CHT)], sem).wait()
  pltpu.sync_copy(rows_v.at[pl.ds(0, CHT)],
                  z_sh.at[dst_v.at[pl.ds(NCH * CH, CHT)]], add=True)

  plsc.subcore_barrier()
  pltpu.sync_copy(z_sh.at[pl.ds(row0, RPS)],
                  out_hbm.at[c, pl.ds(row0, RPS)])


def _xw_body(x_ref, w_ref, xw_ref):
  xw_ref[...] = jnp.dot(x_ref[...], w_ref[...],
                        preferred_element_type=jnp.float32)


def _xw_call(x_p, w):
  return pl.pallas_call(
      _xw_body,
      out_shape=jax.ShapeDtypeStruct((NP, H), jnp.float32),
  )(x_p, w)


def _scale_body(xw_ref, deg_ref, y_ref, dinv_ref):
  deg = deg_ref[0, :, 0] + deg_ref[1, :, 0] + 1.0
  dinv = lax.rsqrt(deg)
  y_ref[...] = xw_ref[...] * dinv[:, None]
  dinv_ref[...] = dinv[:, None]


def _scale_call(xw, deg_parts):
  return pl.pallas_call(
      _scale_body,
      out_shape=(jax.ShapeDtypeStruct((NP, H), jnp.float32),
                 jax.ShapeDtypeStruct((NP, 1), jnp.float32)),
  )(xw, deg_parts)


def _head_body(z_ref, y_ref, dinv_ref, bg_ref, w1_ref, b1_ref, w2_ref, b2_ref,
               w3_ref, b3_ref, o_ref):
  z = z_ref[0] + z_ref[1] + y_ref[...]
  h = jax.nn.relu(z * dinv_ref[...] + bg_ref[...])
  h = jax.nn.relu(
      jnp.dot(h, w1_ref[...], preferred_element_type=jnp.float32) + b1_ref[...])
  h = jax.nn.relu(
      jnp.dot(h, w2_ref[...], preferred_element_type=jnp.float32) + b2_ref[...])
  h = jnp.dot(h, w3_ref[...], preferred_element_type=jnp.float32) + b3_ref[...]
  m = jnp.max(h, axis=1, keepdims=True)
  lse = jnp.log(jnp.sum(jnp.exp(h - m), axis=1, keepdims=True))
  o_ref[...] = h - m - lse


def _head_call(z_parts, y, dinv, bg, w1, b1, w2, b2, w3, b3):
  return pl.pallas_call(
      _head_body,
      out_shape=jax.ShapeDtypeStruct((NP, C), jnp.float32),
  )(z_parts, y, dinv, bg, w1, b1, w2, b2, w3, b3)


def kernel(x, edge_index, W_gcn, b_gcn, W1, b1, W2, b2, W3, b3):
  x_p = jnp.pad(x, ((0, NP - N), (0, 0)))

  deg_parts = _deg_kernel(edge_index)
  xw = _xw_call(x_p, W_gcn)
  y, dinv = _scale_call(xw, deg_parts)
  z_parts = _msg_kernel(y, edge_index)
  out = _head_call(z_parts, y, dinv,
                   b_gcn.reshape(1, H), W1, b1.reshape(1, 32),
                   W2, b2.reshape(1, 16), W3, b3.reshape(1, C))
  return out[:N]


# speedup vs baseline: 2.4286x; 1.3079x over previous
import functools

import jax
import jax.numpy as jnp
from jax import lax
from jax.experimental import pallas as pl
from jax.experimental.pallas import tpu as pltpu, tpu_sc as plsc

N = 10000
E = 320000
D = 128
H = 64
C = 4

NC = 2
NS = 16
NW = NC * NS
NP = 10112
RPS = NP // NS

EPW = E // NW
CH = 128
NCH = EPW // CH
CHT = EPW - NCH * CH
DW = 16

_mesh = plsc.VectorSubcoreMesh(core_axis_name="c", subcore_axis_name="s")
_sc_params = pltpu.CompilerParams(use_tc_tiling_on_sc=False)


def _zero_rows(buf, rows, width):
  zeros16 = jnp.zeros((16,), jnp.float32)

  @pl.loop(0, rows)
  def _(i):
    for k in range(width // 16):
      buf[i, pl.ds(k * 16, 16)] = zeros16


def _init_shared(zbuf, shared, row0):
  for t in range(RPS // CH):
    pltpu.sync_copy(zbuf, shared.at[pl.ds(row0 + t * CH, CH)])
  rem = RPS - (RPS // CH) * CH
  pltpu.sync_copy(zbuf.at[pl.ds(0, rem)],
                  shared.at[pl.ds(row0 + (RPS // CH) * CH, rem)])


@functools.partial(
    pl.kernel,
    out_type=jax.ShapeDtypeStruct((NC, NP, DW), jnp.float32),
    mesh=_mesh,
    compiler_params=_sc_params,
    scratch_types=[
        pltpu.VMEM((EPW,), jnp.int32),
        pltpu.VMEM((CH, DW), jnp.float32),
        pltpu.VMEM((CH, DW), jnp.float32),
        pltpu.VMEM_SHARED((NP, DW), jnp.float32),
    ],
)
def _deg_kernel(ei_hbm, out_hbm, dst_v, ones_v, zbuf, deg_sh):
  c = lax.axis_index("c")
  s = lax.axis_index("s")
  wid = s * NC + c
  row0 = pl.multiple_of(s * RPS, 8)
  base = pl.multiple_of(wid * EPW, 8)

  ones16 = jnp.ones((16,), jnp.float32)

  @pl.loop(0, CH)
  def _(i):
    ones_v[i, pl.ds(0, DW)] = ones16

  _zero_rows(zbuf, CH, DW)
  _init_shared(zbuf, deg_sh, row0)
  pltpu.sync_copy(ei_hbm.at[1, pl.ds(base, EPW)], dst_v)

  plsc.subcore_barrier()

  @pl.loop(0, NCH)
  def _(j):
    pltpu.sync_copy(ones_v, deg_sh.at[dst_v.at[pl.ds(j * CH, CH)]], add=True)

  pltpu.sync_copy(ones_v.at[pl.ds(0, CHT)],
                  deg_sh.at[dst_v.at[pl.ds(NCH * CH, CHT)]], add=True)

  plsc.subcore_barrier()
  pltpu.sync_copy(deg_sh.at[pl.ds(row0, RPS)],
                  out_hbm.at[c, pl.ds(row0, RPS)])


@functools.partial(
    pl.kernel,
    out_type=jax.ShapeDtypeStruct((NC, NP, H), jnp.float32),
    mesh=_mesh,
    compiler_params=_sc_params,
    scratch_types=[
        pltpu.VMEM((EPW,), jnp.int32),
        pltpu.VMEM((EPW,), jnp.int32),
        pltpu.VMEM((CH, H), jnp.float32),
        pltpu.VMEM((CH, H), jnp.float32),
        pltpu.VMEM((CH, H), jnp.float32),
        pltpu.VMEM_SHARED((NP, H), jnp.float32),
        pltpu.SemaphoreType.DMA,
        pltpu.SemaphoreType.DMA,
    ],
)
def _msg_kernel(y_hbm, ei_hbm, out_hbm, src_v, dst_v, rows_a, rows_b, zbuf,
                z_sh, sem_a, sem_b):
  c = lax.axis_index("c")
  s = lax.axis_index("s")
  wid = s * NC + c
  row0 = pl.multiple_of(s * RPS, 8)
  base = pl.multiple_of(wid * EPW, 8)

  _zero_rows(zbuf, CH, H)
  _init_shared(zbuf, z_sh, row0)
  pltpu.sync_copy(ei_hbm.at[0, pl.ds(base, EPW)], src_v)
  pltpu.sync_copy(ei_hbm.at[1, pl.ds(base, EPW)], dst_v)
  plsc.subcore_barrier()

  pltpu.async_copy(y_hbm.at[src_v.at[pl.ds(0, CH)]], rows_a, sem_a)

  @pl.loop(0, NCH // 2)
  def _(g):
    j0 = g * 2
    pltpu.async_copy(
        y_hbm.at[src_v.at[pl.ds((j0 + 1) * CH, CH)]], rows_b, sem_b)
    pltpu.make_async_copy(
        y_hbm.at[src_v.at[pl.ds(0, CH)]], rows_a, sem_a).wait()
    pltpu.sync_copy(rows_a, z_sh.at[dst_v.at[pl.ds(j0 * CH, CH)]], add=True)
    nxt = pl.multiple_of(
        jnp.minimum((j0 + 2) * CH, (NCH - 1) * CH), 8)
    pltpu.async_copy(y_hbm.at[src_v.at[pl.ds(nxt, CH)]], rows_a, sem_a)
    pltpu.make_async_copy(
        y_hbm.at[src_v.at[pl.ds(0, CH)]], rows_b, sem_b).wait()
    pltpu.sync_copy(rows_b, z_sh.at[dst_v.at[pl.ds((j0 + 1) * CH, CH)]],
                    add=True)

  pltpu.make_async_copy(
      y_hbm.at[src_v.at[pl.ds(0, CH)]], rows_a, sem_a).wait()
  pltpu.async_copy(
      y_hbm.at[src_v.at[pl.ds(NCH * CH, CHT)]],
      rows_a.at[pl.ds(0, CHT)], sem_a).wait()
  pltpu.sync_copy(rows_a.at[pl.ds(0, CHT)],
                  z_sh.at[dst_v.at[pl.ds(NCH * CH, CHT)]], add=True)

  plsc.subcore_barrier()
  pltpu.sync_copy(z_sh.at[pl.ds(row0, RPS)],
                  out_hbm.at[c, pl.ds(row0, RPS)])


def _mm_body(x_ref, w_ref, deg_ref, y_ref, dinv_ref):
  deg = deg_ref[0, :, 0] + deg_ref[1, :, 0] + 1.0
  dinv = lax.rsqrt(deg)
  xw = jnp.dot(x_ref[...], w_ref[...], preferred_element_type=jnp.float32)
  y_ref[...] = xw * dinv[:, None]
  dinv_ref[...] = dinv[:, None]


def _mm_call(x_p, w, deg_parts):
  return pl.pallas_call(
      _mm_body,
      out_shape=(jax.ShapeDtypeStruct((NP, H), jnp.float32),
                 jax.ShapeDtypeStruct((NP, 1), jnp.float32)),
  )(x_p, w, deg_parts)


def _head_body(z_ref, y_ref, dinv_ref, bg_ref, w1_ref, b1_ref, w2_ref, b2_ref,
               w3_ref, b3_ref, o_ref):
  z = z_ref[0] + z_ref[1] + y_ref[...]
  h = jax.nn.relu(z * dinv_ref[...] + bg_ref[...])
  h = jax.nn.relu(
      jnp.dot(h, w1_ref[...], preferred_element_type=jnp.float32) + b1_ref[...])
  h = jax.nn.relu(
      jnp.dot(h, w2_ref[...], preferred_element_type=jnp.float32) + b2_ref[...])
  h = jnp.dot(h, w3_ref[...], preferred_element_type=jnp.float32) + b3_ref[...]
  m = jnp.max(h, axis=1, keepdims=True)
  lse = jnp.log(jnp.sum(jnp.exp(h - m), axis=1, keepdims=True))
  o_ref[...] = h - m - lse


def _head_call(z_parts, y, dinv, bg, w1, b1, w2, b2, w3, b3):
  return pl.pallas_call(
      _head_body,
      out_shape=jax.ShapeDtypeStruct((NP, C), jnp.float32),
  )(z_parts, y, dinv, bg, w1, b1, w2, b2, w3, b3)


def kernel(x, edge_index, W_gcn, b_gcn, W1, b1, W2, b2, W3, b3):
  x_p = jnp.pad(x, ((0, NP - N), (0, 0)))

  deg_parts = _deg_kernel(edge_index)
  y, dinv = _mm_call(x_p, W_gcn, deg_parts)
  z_parts = _msg_kernel(y, edge_index)
  out = _head_call(z_parts, y, dinv,
                   b_gcn.reshape(1, H), W1, b1.reshape(1, 32),
                   W2, b2.reshape(1, 16), W3, b3.reshape(1, C))
  return out[:N]


# speedup vs baseline: 2.4743x; 1.0188x over previous
import functools

import jax
import jax.numpy as jnp
from jax import lax
from jax.experimental import pallas as pl
from jax.experimental.pallas import tpu as pltpu, tpu_sc as plsc

N = 10000
E = 320000
D = 128
H = 64
C = 4

NC = 2
NS = 16
NW = NC * NS
NP = 10112
RPS = NP // NS

EPW = E // NW
CH = 128
NCH = EPW // CH
CHT = EPW - NCH * CH
DW = 16

_mesh = plsc.VectorSubcoreMesh(core_axis_name="c", subcore_axis_name="s")
_sc_params = pltpu.CompilerParams(use_tc_tiling_on_sc=False)


def _zero_rows(buf, rows, width):
  zeros16 = jnp.zeros((16,), jnp.float32)

  @pl.loop(0, rows)
  def _(i):
    for k in range(width // 16):
      buf[i, pl.ds(k * 16, 16)] = zeros16


def _init_shared(zbuf, shared, row0):
  for t in range(RPS // CH):
    pltpu.sync_copy(zbuf, shared.at[pl.ds(row0 + t * CH, CH)])
  rem = RPS - (RPS // CH) * CH
  pltpu.sync_copy(zbuf.at[pl.ds(0, rem)],
                  shared.at[pl.ds(row0 + (RPS // CH) * CH, rem)])


@functools.partial(
    pl.kernel,
    out_type=jax.ShapeDtypeStruct((NC, NP, DW), jnp.float32),
    mesh=_mesh,
    compiler_params=_sc_params,
    scratch_types=[
        pltpu.VMEM((EPW,), jnp.int32),
        pltpu.VMEM((CH, DW), jnp.float32),
        pltpu.VMEM((CH, DW), jnp.float32),
        pltpu.VMEM_SHARED((NP, DW), jnp.float32),
    ],
)
def _deg_kernel(ei_hbm, out_hbm, dst_v, ones_v, zbuf, deg_sh):
  c = lax.axis_index("c")
  s = lax.axis_index("s")
  wid = s * NC + c
  row0 = pl.multiple_of(s * RPS, 8)
  base = pl.multiple_of(wid * EPW, 8)

  ones16 = jnp.ones((16,), jnp.float32)

  @pl.loop(0, CH)
  def _(i):
    ones_v[i, pl.ds(0, DW)] = ones16

  _zero_rows(zbuf, CH, DW)
  _init_shared(zbuf, deg_sh, row0)
  pltpu.sync_copy(ei_hbm.at[1, pl.ds(base, EPW)], dst_v)

  plsc.subcore_barrier()

  @pl.loop(0, NCH)
  def _(j):
    pltpu.sync_copy(ones_v, deg_sh.at[dst_v.at[pl.ds(j * CH, CH)]], add=True)

  pltpu.sync_copy(ones_v.at[pl.ds(0, CHT)],
                  deg_sh.at[dst_v.at[pl.ds(NCH * CH, CHT)]], add=True)

  plsc.subcore_barrier()
  pltpu.sync_copy(deg_sh.at[pl.ds(row0, RPS)],
                  out_hbm.at[c, pl.ds(row0, RPS)])


@functools.partial(
    pl.kernel,
    out_type=jax.ShapeDtypeStruct((NC, NP, H), jnp.float32),
    mesh=_mesh,
    compiler_params=_sc_params,
    scratch_types=[
        pltpu.VMEM((EPW,), jnp.int32),
        pltpu.VMEM((EPW,), jnp.int32),
        pltpu.VMEM((CH, H), jnp.float32),
        pltpu.VMEM((CH, H), jnp.float32),
        pltpu.VMEM((CH, H), jnp.float32),
        pltpu.VMEM_SHARED((NP, H), jnp.float32),
        pltpu.SemaphoreType.DMA,
        pltpu.SemaphoreType.DMA,
    ],
)
def _msg_kernel(y_hbm, ei_hbm, out_hbm, src_v, dst_v, rows_a, rows_b, zbuf,
                z_sh, sem_a, sem_b):
  c = lax.axis_index("c")
  s = lax.axis_index("s")
  wid = s * NC + c
  row0 = pl.multiple_of(s * RPS, 8)
  base = pl.multiple_of(wid * EPW, 8)

  _zero_rows(zbuf, CH, H)
  _init_shared(zbuf, z_sh, row0)
  pltpu.sync_copy(ei_hbm.at[0, pl.ds(base, EPW)], src_v)
  pltpu.sync_copy(ei_hbm.at[1, pl.ds(base, EPW)], dst_v)
  plsc.subcore_barrier()

  pltpu.async_copy(y_hbm.at[src_v.at[pl.ds(0, CH)]], rows_a, sem_a)

  @pl.loop(0, NCH // 2)
  def _(g):
    j0 = g * 2
    pltpu.async_copy(
        y_hbm.at[src_v.at[pl.ds((j0 + 1) * CH, CH)]], rows_b, sem_b)
    pltpu.make_async_copy(
        y_hbm.at[src_v.at[pl.ds(0, CH)]], rows_a, sem_a).wait()
    pltpu.sync_copy(rows_a, z_sh.at[dst_v.at[pl.ds(j0 * CH, CH)]], add=True)
    nxt = pl.multiple_of(
        jnp.minimum((j0 + 2) * CH, (NCH - 1) * CH), 8)
    pltpu.async_copy(y_hbm.at[src_v.at[pl.ds(nxt, CH)]], rows_a, sem_a)
    pltpu.make_async_copy(
        y_hbm.at[src_v.at[pl.ds(0, CH)]], rows_b, sem_b).wait()
    pltpu.sync_copy(rows_b, z_sh.at[dst_v.at[pl.ds((j0 + 1) * CH, CH)]],
                    add=True)

  pltpu.make_async_copy(
      y_hbm.at[src_v.at[pl.ds(0, CH)]], rows_a, sem_a).wait()
  pltpu.async_copy(
      y_hbm.at[src_v.at[pl.ds(NCH * CH, CHT)]],
      rows_a.at[pl.ds(0, CHT)], sem_a).wait()
  pltpu.sync_copy(rows_a.at[pl.ds(0, CHT)],
                  z_sh.at[dst_v.at[pl.ds(NCH * CH, CHT)]], add=True)

  plsc.subcore_barrier()
  pltpu.sync_copy(z_sh.at[pl.ds(row0, RPS)],
                  out_hbm.at[c, pl.ds(row0, RPS)])


def _mm_body(x_ref, w_ref, deg_ref, y_ref, dinv_ref):
  deg = deg_ref[0, pl.ds(0, N), 0] + deg_ref[1, pl.ds(0, N), 0] + 1.0
  dinv = lax.rsqrt(deg)
  xw = jnp.dot(x_ref[...], w_ref[...], preferred_element_type=jnp.float32)
  y_ref[...] = xw * dinv[:, None]
  dinv_ref[...] = dinv[:, None]


def _mm_call(x, w, deg_parts):
  return pl.pallas_call(
      _mm_body,
      out_shape=(jax.ShapeDtypeStruct((N, H), jnp.float32),
                 jax.ShapeDtypeStruct((N, 1), jnp.float32)),
  )(x, w, deg_parts)


def _head_body(z_ref, y_ref, dinv_ref, bg_ref, w1_ref, b1_ref, w2_ref, b2_ref,
               w3_ref, b3_ref, o_ref):
  z = z_ref[0, pl.ds(0, N), :] + z_ref[1, pl.ds(0, N), :] + y_ref[...]
  h = jax.nn.relu(z * dinv_ref[...] + bg_ref[...])
  h = jax.nn.relu(
      jnp.dot(h, w1_ref[...], preferred_element_type=jnp.float32) + b1_ref[...])
  h = jax.nn.relu(
      jnp.dot(h, w2_ref[...], preferred_element_type=jnp.float32) + b2_ref[...])
  h = jnp.dot(h, w3_ref[...], preferred_element_type=jnp.float32) + b3_ref[...]
  m = jnp.max(h, axis=1, keepdims=True)
  lse = jnp.log(jnp.sum(jnp.exp(h - m), axis=1, keepdims=True))
  o_ref[...] = h - m - lse


def _head_call(z_parts, y, dinv, bg, w1, b1, w2, b2, w3, b3):
  return pl.pallas_call(
      _head_body,
      out_shape=jax.ShapeDtypeStruct((N, C), jnp.float32),
  )(z_parts, y, dinv, bg, w1, b1, w2, b2, w3, b3)


def kernel(x, edge_index, W_gcn, b_gcn, W1, b1, W2, b2, W3, b3):
  deg_parts = _deg_kernel(edge_index)
  y, dinv = _mm_call(x, W_gcn, deg_parts)
  z_parts = _msg_kernel(y, edge_index)
  return _head_call(z_parts, y, dinv,
                    b_gcn.reshape(1, H), W1, b1.reshape(1, 32),
                    W2, b2.reshape(1, 16), W3, b3.reshape(1, C))


# speedup vs baseline: 2.5084x; 1.0138x over previous
import functools

import jax
import jax.numpy as jnp
from jax import lax
from jax.experimental import pallas as pl
from jax.experimental.pallas import tpu as pltpu, tpu_sc as plsc

N = 10000
E = 320000
D = 128
H = 64
C = 4

NC = 2
NS = 16
NW = NC * NS
NP = 10112
RPS = NP // NS

EPW = E // NW
CH = 128
NCH = EPW // CH
CHT = EPW - NCH * CH
DW = 16

_mesh = plsc.VectorSubcoreMesh(core_axis_name="c", subcore_axis_name="s")
_sc_params = pltpu.CompilerParams(use_tc_tiling_on_sc=False)


def _zero_rows(buf, rows, width):
  zeros16 = jnp.zeros((16,), jnp.float32)

  @pl.loop(0, rows)
  def _(i):
    for k in range(width // 16):
      buf[i, pl.ds(k * 16, 16)] = zeros16


def _init_shared(zbuf, shared, row0):
  for t in range(RPS // CH):
    pltpu.sync_copy(zbuf, shared.at[pl.ds(row0 + t * CH, CH)])
  rem = RPS - (RPS // CH) * CH
  pltpu.sync_copy(zbuf.at[pl.ds(0, rem)],
                  shared.at[pl.ds(row0 + (RPS // CH) * CH, rem)])


@functools.partial(
    pl.kernel,
    out_type=jax.ShapeDtypeStruct((NC, NP, DW), jnp.float32),
    mesh=_mesh,
    compiler_params=_sc_params,
    scratch_types=[
        pltpu.VMEM((EPW,), jnp.int32),
        pltpu.VMEM((CH, DW), jnp.float32),
        pltpu.VMEM((CH, DW), jnp.float32),
        pltpu.VMEM_SHARED((NP, DW), jnp.float32),
        pltpu.SemaphoreType.DMA,
        pltpu.SemaphoreType.DMA,
    ],
)
def _deg_kernel(ei_hbm, out_hbm, dst_v, ones_v, zbuf, deg_sh, sem_a, sem_b):
  c = lax.axis_index("c")
  s = lax.axis_index("s")
  wid = s * NC + c
  row0 = pl.multiple_of(s * RPS, 8)
  base = pl.multiple_of(E + wid * EPW, 8)

  ones16 = jnp.ones((16,), jnp.float32)

  @pl.loop(0, CH)
  def _(i):
    ones_v[i, pl.ds(0, DW)] = ones16

  _zero_rows(zbuf, CH, DW)
  _init_shared(zbuf, deg_sh, row0)
  pltpu.sync_copy(ei_hbm.at[pl.ds(base, EPW)], dst_v)

  plsc.subcore_barrier()

  def _add(j, sem):
    return pltpu.async_copy(
        ones_v, deg_sh.at[dst_v.at[pl.ds(j * CH, CH)]], sem, add=True)

  def _wait(sem):
    pltpu.make_async_copy(
        ones_v, deg_sh.at[dst_v.at[pl.ds(0, CH)]], sem).wait()

  _add(0, sem_a)
  _add(1, sem_b)

  @pl.loop(0, NCH // 2 - 1)
  def _(g):
    _wait(sem_a)
    _add(g * 2 + 2, sem_a)
    _wait(sem_b)
    _add(g * 2 + 3, sem_b)

  _wait(sem_a)
  _wait(sem_b)
  pltpu.sync_copy(ones_v.at[pl.ds(0, CHT)],
                  deg_sh.at[dst_v.at[pl.ds(NCH * CH, CHT)]], add=True)

  plsc.subcore_barrier()
  pltpu.sync_copy(deg_sh.at[pl.ds(row0, RPS)],
                  out_hbm.at[c, pl.ds(row0, RPS)])


@functools.partial(
    pl.kernel,
    out_type=jax.ShapeDtypeStruct((NC, NP, H), jnp.float32),
    mesh=_mesh,
    compiler_params=_sc_params,
    scratch_types=[
        pltpu.VMEM((EPW,), jnp.int32),
        pltpu.VMEM((EPW,), jnp.int32),
        pltpu.VMEM((CH, H), jnp.float32),
        pltpu.VMEM((CH, H), jnp.float32),
        pltpu.VMEM((CH, H), jnp.float32),
        pltpu.VMEM_SHARED((NP, H), jnp.float32),
        pltpu.SemaphoreType.DMA,
        pltpu.SemaphoreType.DMA,
    ],
)
def _msg_kernel(y_hbm, ei_hbm, out_hbm, src_v, dst_v, rows_a, rows_b, zbuf,
                z_sh, sem_a, sem_b):
  c = lax.axis_index("c")
  s = lax.axis_index("s")
  wid = s * NC + c
  row0 = pl.multiple_of(s * RPS, 8)
  base = pl.multiple_of(wid * EPW, 8)

  _zero_rows(zbuf, CH, H)
  _init_shared(zbuf, z_sh, row0)
  pltpu.sync_copy(ei_hbm.at[pl.ds(base, EPW)], src_v)
  pltpu.sync_copy(ei_hbm.at[pl.ds(pl.multiple_of(E + base, 8), EPW)], dst_v)
  plsc.subcore_barrier()

  pltpu.async_copy(y_hbm.at[src_v.at[pl.ds(0, CH)]], rows_a, sem_a)

  @pl.loop(0, NCH // 2)
  def _(g):
    j0 = g * 2
    pltpu.async_copy(
        y_hbm.at[src_v.at[pl.ds((j0 + 1) * CH, CH)]], rows_b, sem_b)
    pltpu.make_async_copy(
        y_hbm.at[src_v.at[pl.ds(0, CH)]], rows_a, sem_a).wait()
    pltpu.sync_copy(rows_a, z_sh.at[dst_v.at[pl.ds(j0 * CH, CH)]], add=True)
    nxt = pl.multiple_of(
        jnp.minimum((j0 + 2) * CH, (NCH - 1) * CH), 8)
    pltpu.async_copy(y_hbm.at[src_v.at[pl.ds(nxt, CH)]], rows_a, sem_a)
    pltpu.make_async_copy(
        y_hbm.at[src_v.at[pl.ds(0, CH)]], rows_b, sem_b).wait()
    pltpu.sync_copy(rows_b, z_sh.at[dst_v.at[pl.ds((j0 + 1) * CH, CH)]],
                    add=True)

  pltpu.make_async_copy(
      y_hbm.at[src_v.at[pl.ds(0, CH)]], rows_a, sem_a).wait()
  pltpu.async_copy(
      y_hbm.at[src_v.at[pl.ds(NCH * CH, CHT)]],
      rows_a.at[pl.ds(0, CHT)], sem_a).wait()
  pltpu.sync_copy(rows_a.at[pl.ds(0, CHT)],
                  z_sh.at[dst_v.at[pl.ds(NCH * CH, CHT)]], add=True)

  plsc.subcore_barrier()
  pltpu.sync_copy(z_sh.at[pl.ds(row0, RPS)],
                  out_hbm.at[c, pl.ds(row0, RPS)])


def _mm_body(x_ref, w_ref, deg_ref, y_ref, dinv_ref):
  deg = deg_ref[0, pl.ds(0, N), 0] + deg_ref[1, pl.ds(0, N), 0] + 1.0
  dinv = lax.rsqrt(deg)
  xw = jnp.dot(x_ref[...], w_ref[...], preferred_element_type=jnp.float32)
  y_ref[...] = xw * dinv[:, None]
  dinv_ref[...] = dinv[:, None]


def _mm_call(x, w, deg_parts):
  return pl.pallas_call(
      _mm_body,
      out_shape=(jax.ShapeDtypeStruct((N, H), jnp.float32),
                 jax.ShapeDtypeStruct((N, 1), jnp.float32)),
  )(x, w, deg_parts)


def _head_body(z_ref, y_ref, dinv_ref, bg_ref, w1_ref, b1_ref, w2_ref, b2_ref,
               w3_ref, b3_ref, o_ref):
  z = z_ref[0, pl.ds(0, N), :] + z_ref[1, pl.ds(0, N), :] + y_ref[...]
  h = jax.nn.relu(z * dinv_ref[...] + bg_ref[...])
  h = jax.nn.relu(
      jnp.dot(h, w1_ref[...], preferred_element_type=jnp.float32) + b1_ref[...])
  h = jax.nn.relu(
      jnp.dot(h, w2_ref[...], preferred_element_type=jnp.float32) + b2_ref[...])
  h = jnp.dot(h, w3_ref[...], preferred_element_type=jnp.float32) + b3_ref[...]
  m = jnp.max(h, axis=1, keepdims=True)
  lse = jnp.log(jnp.sum(jnp.exp(h - m), axis=1, keepdims=True))
  o_ref[...] = h - m - lse


def _head_call(z_parts, y, dinv, bg, w1, b1, w2, b2, w3, b3):
  return pl.pallas_call(
      _head_body,
      out_shape=jax.ShapeDtypeStruct((N, C), jnp.float32),
  )(z_parts, y, dinv, bg, w1, b1, w2, b2, w3, b3)


def kernel(x, edge_index, W_gcn, b_gcn, W1, b1, W2, b2, W3, b3):
  ei = edge_index.reshape(-1)
  deg_parts = _deg_kernel(ei)
  y, dinv = _mm_call(x, W_gcn, deg_parts)
  z_parts = _msg_kernel(y, ei)
  return _head_call(z_parts, y, dinv,
                    b_gcn.reshape(1, H), W1, b1.reshape(1, 32),
                    W2, b2.reshape(1, 16), W3, b3.reshape(1, C))


# speedup vs baseline: 2.5710x; 1.0250x over previous
import functools

import jax
import jax.numpy as jnp
from jax import lax
from jax.experimental import pallas as pl
from jax.experimental.pallas import tpu as pltpu, tpu_sc as plsc

N = 10000
E = 320000
D = 128
H = 64
C = 4

NC = 2
NS = 16
NW = NC * NS
NP = 10112
RPS = NP // NS

EPW = E // NW
CH = 128
NCH = EPW // CH
CHT = EPW - NCH * CH
DW = 16

_mesh = plsc.VectorSubcoreMesh(core_axis_name="c", subcore_axis_name="s")
_sc_params = pltpu.CompilerParams(use_tc_tiling_on_sc=False)


def _zero_rows(buf, rows, width):
  zeros16 = jnp.zeros((16,), jnp.float32)

  @pl.loop(0, rows)
  def _(i):
    for k in range(width // 16):
      buf[i, pl.ds(k * 16, 16)] = zeros16


def _init_shared(zbuf, shared, row0):
  for t in range(RPS // CH):
    pltpu.sync_copy(zbuf, shared.at[pl.ds(row0 + t * CH, CH)])
  rem = RPS - (RPS // CH) * CH
  pltpu.sync_copy(zbuf.at[pl.ds(0, rem)],
                  shared.at[pl.ds(row0 + (RPS // CH) * CH, rem)])


@functools.partial(
    pl.kernel,
    out_type=jax.ShapeDtypeStruct((NC, NP, DW), jnp.float32),
    mesh=_mesh,
    compiler_params=_sc_params,
    scratch_types=[
        pltpu.VMEM((EPW,), jnp.int32),
        pltpu.VMEM((CH, DW), jnp.float32),
        pltpu.VMEM((CH, DW), jnp.float32),
        pltpu.VMEM_SHARED((NP, DW), jnp.float32),
        pltpu.SemaphoreType.DMA,
        pltpu.SemaphoreType.DMA,
    ],
)
def _deg_kernel(ei_hbm, out_hbm, dst_v, ones_v, zbuf, deg_sh, sem_a, sem_b):
  c = lax.axis_index("c")
  s = lax.axis_index("s")
  wid = s * NC + c
  row0 = pl.multiple_of(s * RPS, 8)
  base = pl.multiple_of(E + wid * EPW, 8)

  ones16 = jnp.ones((16,), jnp.float32)

  pltpu.async_copy(ei_hbm.at[pl.ds(base, EPW)], dst_v, sem_a)

  @pl.loop(0, CH)
  def _(i):
    ones_v[i, pl.ds(0, DW)] = ones16

  _zero_rows(zbuf, CH, DW)
  _init_shared(zbuf, deg_sh, row0)
  pltpu.make_async_copy(ei_hbm.at[pl.ds(base, EPW)], dst_v, sem_a).wait()

  plsc.subcore_barrier()

  def _add(j, sem):
    return pltpu.async_copy(
        ones_v, deg_sh.at[dst_v.at[pl.ds(j * CH, CH)]], sem, add=True)

  def _wait(sem):
    pltpu.make_async_copy(
        ones_v, deg_sh.at[dst_v.at[pl.ds(0, CH)]], sem).wait()

  _add(0, sem_a)
  _add(1, sem_b)

  @pl.loop(0, NCH // 2 - 1)
  def _(g):
    _wait(sem_a)
    _add(g * 2 + 2, sem_a)
    _wait(sem_b)
    _add(g * 2 + 3, sem_b)

  _wait(sem_a)
  _wait(sem_b)
  pltpu.sync_copy(ones_v.at[pl.ds(0, CHT)],
                  deg_sh.at[dst_v.at[pl.ds(NCH * CH, CHT)]], add=True)

  plsc.subcore_barrier()
  pltpu.sync_copy(deg_sh.at[pl.ds(row0, RPS)],
                  out_hbm.at[c, pl.ds(row0, RPS)])


@functools.partial(
    pl.kernel,
    out_type=jax.ShapeDtypeStruct((NC, NP, H), jnp.float32),
    mesh=_mesh,
    compiler_params=_sc_params,
    scratch_types=[
        pltpu.VMEM((EPW,), jnp.int32),
        pltpu.VMEM((EPW,), jnp.int32),
        pltpu.VMEM((CH, H), jnp.float32),
        pltpu.VMEM((CH, H), jnp.float32),
        pltpu.VMEM((CH, H), jnp.float32),
        pltpu.VMEM_SHARED((NP, H), jnp.float32),
        pltpu.SemaphoreType.DMA,
        pltpu.SemaphoreType.DMA,
    ],
)
def _msg_kernel(y_hbm, ei_hbm, out_hbm, src_v, dst_v, rows_a, rows_b, zbuf,
                z_sh, sem_a, sem_b):
  c = lax.axis_index("c")
  s = lax.axis_index("s")
  wid = s * NC + c
  row0 = pl.multiple_of(s * RPS, 8)
  base = pl.multiple_of(wid * EPW, 8)

  pltpu.async_copy(ei_hbm.at[pl.ds(base, EPW)], src_v, sem_a)
  pltpu.async_copy(
      ei_hbm.at[pl.ds(pl.multiple_of(E + base, 8), EPW)], dst_v, sem_b)
  _zero_rows(zbuf, CH, H)
  _init_shared(zbuf, z_sh, row0)
  pltpu.make_async_copy(ei_hbm.at[pl.ds(base, EPW)], src_v, sem_a).wait()
  pltpu.make_async_copy(ei_hbm.at[pl.ds(base, EPW)], dst_v, sem_b).wait()
  plsc.subcore_barrier()

  pltpu.async_copy(y_hbm.at[src_v.at[pl.ds(0, CH)]], rows_a, sem_a)

  @pl.loop(0, NCH // 2)
  def _(g):
    j0 = g * 2
    pltpu.async_copy(
        y_hbm.at[src_v.at[pl.ds((j0 + 1) * CH, CH)]], rows_b, sem_b)
    pltpu.make_async_copy(
        y_hbm.at[src_v.at[pl.ds(0, CH)]], rows_a, sem_a).wait()
    pltpu.sync_copy(rows_a, z_sh.at[dst_v.at[pl.ds(j0 * CH, CH)]], add=True)
    nxt = pl.multiple_of(
        jnp.minimum((j0 + 2) * CH, (NCH - 1) * CH), 8)
    pltpu.async_copy(y_hbm.at[src_v.at[pl.ds(nxt, CH)]], rows_a, sem_a)
    pltpu.make_async_copy(
        y_hbm.at[src_v.at[pl.ds(0, CH)]], rows_b, sem_b).wait()
    pltpu.sync_copy(rows_b, z_sh.at[dst_v.at[pl.ds((j0 + 1) * CH, CH)]],
                    add=True)

  pltpu.make_async_copy(
      y_hbm.at[src_v.at[pl.ds(0, CH)]], rows_a, sem_a).wait()
  pltpu.async_copy(
      y_hbm.at[src_v.at[pl.ds(NCH * CH, CHT)]],
      rows_a.at[pl.ds(0, CHT)], sem_a).wait()
  pltpu.sync_copy(rows_a.at[pl.ds(0, CHT)],
                  z_sh.at[dst_v.at[pl.ds(NCH * CH, CHT)]], add=True)

  plsc.subcore_barrier()
  pltpu.sync_copy(z_sh.at[pl.ds(row0, RPS)],
                  out_hbm.at[c, pl.ds(row0, RPS)])


def _mm_body(x_ref, w_ref, deg_ref, y_ref, dinv_ref):
  deg = deg_ref[0, pl.ds(0, N), 0] + deg_ref[1, pl.ds(0, N), 0] + 1.0
  dinv = lax.rsqrt(deg)
  xw = jnp.dot(x_ref[...], w_ref[...], preferred_element_type=jnp.float32)
  y_ref[...] = xw * dinv[:, None]
  dinv_ref[...] = dinv[:, None]


def _mm_call(x, w, deg_parts):
  return pl.pallas_call(
      _mm_body,
      out_shape=(jax.ShapeDtypeStruct((N, H), jnp.float32),
                 jax.ShapeDtypeStruct((N, 1), jnp.float32)),
  )(x, w, deg_parts)


def _head_body(z_ref, y_ref, dinv_ref, bg_ref, w1_ref, b1_ref, w2_ref, b2_ref,
               w3_ref, b3_ref, o_ref):
  z = z_ref[0, pl.ds(0, N), :] + z_ref[1, pl.ds(0, N), :] + y_ref[...]
  h = jax.nn.relu(z * dinv_ref[...] + bg_ref[...])
  h = jax.nn.relu(
      jnp.dot(h, w1_ref[...], preferred_element_type=jnp.float32) + b1_ref[...])
  h = jax.nn.relu(
      jnp.dot(h, w2_ref[...], preferred_element_type=jnp.float32) + b2_ref[...])
  h = jnp.dot(h, w3_ref[...], preferred_element_type=jnp.float32) + b3_ref[...]
  m = jnp.max(h, axis=1, keepdims=True)
  lse = jnp.log(jnp.sum(jnp.exp(h - m), axis=1, keepdims=True))
  o_ref[...] = h - m - lse


def _head_call(z_parts, y, dinv, bg, w1, b1, w2, b2, w3, b3):
  return pl.pallas_call(
      _head_body,
      out_shape=jax.ShapeDtypeStruct((N, C), jnp.float32),
  )(z_parts, y, dinv, bg, w1, b1, w2, b2, w3, b3)


def kernel(x, edge_index, W_gcn, b_gcn, W1, b1, W2, b2, W3, b3):
  ei = edge_index.reshape(-1)
  deg_parts = _deg_kernel(ei)
  y, dinv = _mm_call(x, W_gcn, deg_parts)
  z_parts = _msg_kernel(y, ei)
  return _head_call(z_parts, y, dinv,
                    b_gcn.reshape(1, H), W1, b1.reshape(1, 32),
                    W2, b2.reshape(1, 16), W3, b3.reshape(1, C))


# speedup vs baseline: 2.8236x; 1.0982x over previous
import functools

import jax
import jax.numpy as jnp
from jax import lax
from jax.experimental import pallas as pl
from jax.experimental.pallas import tpu as pltpu, tpu_sc as plsc

N = 10000
E = 320000
D = 128
H = 64
C = 4

NC = 2
NS = 16
NW = NC * NS
NP = 10112
RPS = NP // NS

EPW = E // NW
CH = 128
NCH = EPW // CH
CHT = EPW - NCH * CH
DW = 16

_mesh = plsc.VectorSubcoreMesh(core_axis_name="c", subcore_axis_name="s")
_sc_params = pltpu.CompilerParams(use_tc_tiling_on_sc=False)


def _zero_rows(buf, rows, width):
  zeros16 = jnp.zeros((16,), jnp.float32)

  @pl.loop(0, rows)
  def _(i):
    for k in range(width // 16):
      buf[i, pl.ds(k * 16, 16)] = zeros16


def _init_shared(zbuf, shared, row0):
  for t in range(RPS // CH):
    pltpu.sync_copy(zbuf, shared.at[pl.ds(row0 + t * CH, CH)])
  rem = RPS - (RPS // CH) * CH
  pltpu.sync_copy(zbuf.at[pl.ds(0, rem)],
                  shared.at[pl.ds(row0 + (RPS // CH) * CH, rem)])


@functools.partial(
    pl.kernel,
    out_type=jax.ShapeDtypeStruct((NC, NP, DW), jnp.float32),
    mesh=_mesh,
    compiler_params=_sc_params,
    scratch_types=[
        pltpu.VMEM((EPW,), jnp.int32),
        pltpu.VMEM((CH, DW), jnp.float32),
        pltpu.VMEM((CH, DW), jnp.float32),
        pltpu.VMEM_SHARED((NP, DW), jnp.float32),
        pltpu.SemaphoreType.DMA,
        pltpu.SemaphoreType.DMA,
    ],
)
def _deg_kernel(ei_hbm, out_hbm, dst_v, ones_v, zbuf, deg_sh, sem_a, sem_b):
  c = lax.axis_index("c")
  s = lax.axis_index("s")
  wid = s * NC + c
  row0 = pl.multiple_of(s * RPS, 8)
  base = pl.multiple_of(E + wid * EPW, 8)

  ones16 = jnp.ones((16,), jnp.float32)

  pltpu.async_copy(ei_hbm.at[pl.ds(base, EPW)], dst_v, sem_a)

  @pl.loop(0, CH)
  def _(i):
    ones_v[i, pl.ds(0, DW)] = ones16

  _zero_rows(zbuf, CH, DW)
  _init_shared(zbuf, deg_sh, row0)
  pltpu.make_async_copy(ei_hbm.at[pl.ds(base, EPW)], dst_v, sem_a).wait()

  plsc.subcore_barrier()

  def _add(j, sem):
    return pltpu.async_copy(
        ones_v, deg_sh.at[dst_v.at[pl.ds(j * CH, CH)]], sem, add=True)

  def _wait(sem):
    pltpu.make_async_copy(
        ones_v, deg_sh.at[dst_v.at[pl.ds(0, CH)]], sem).wait()

  _add(0, sem_a)
  _add(1, sem_b)

  @pl.loop(0, NCH // 2 - 1)
  def _(g):
    _wait(sem_a)
    _add(g * 2 + 2, sem_a)
    _wait(sem_b)
    _add(g * 2 + 3, sem_b)

  _wait(sem_a)
  _wait(sem_b)
  pltpu.sync_copy(ones_v.at[pl.ds(0, CHT)],
                  deg_sh.at[dst_v.at[pl.ds(NCH * CH, CHT)]], add=True)

  plsc.subcore_barrier()
  pltpu.sync_copy(deg_sh.at[pl.ds(row0, RPS)],
                  out_hbm.at[c, pl.ds(row0, RPS)])


@functools.partial(
    pl.kernel,
    out_type=jax.ShapeDtypeStruct((NC, NP, H), jnp.float32),
    mesh=_mesh,
    compiler_params=_sc_params,
    scratch_types=[
        pltpu.VMEM((EPW,), jnp.int32),
        pltpu.VMEM((EPW,), jnp.int32),
        pltpu.VMEM((CH, H), jnp.float32),
        pltpu.VMEM((CH, H), jnp.float32),
        pltpu.VMEM((CH, H), jnp.float32),
        pltpu.VMEM((CH, H), jnp.float32),
        pltpu.VMEM_SHARED((NP, H), jnp.float32),
        pltpu.SemaphoreType.DMA,
        pltpu.SemaphoreType.DMA,
        pltpu.SemaphoreType.DMA,
    ],
)
def _msg_kernel(y_hbm, ei_hbm, out_hbm, src_v, dst_v, rows_a, rows_b, rows_c,
                zbuf, z_sh, sem_a, sem_b, sem_c):
  c = lax.axis_index("c")
  s = lax.axis_index("s")
  wid = s * NC + c
  row0 = pl.multiple_of(s * RPS, 8)
  base = pl.multiple_of(wid * EPW, 8)

  pltpu.async_copy(ei_hbm.at[pl.ds(base, EPW)], src_v, sem_a)
  pltpu.async_copy(
      ei_hbm.at[pl.ds(pl.multiple_of(E + base, 8), EPW)], dst_v, sem_b)
  _zero_rows(zbuf, CH, H)
  _init_shared(zbuf, z_sh, row0)
  pltpu.make_async_copy(ei_hbm.at[pl.ds(base, EPW)], src_v, sem_a).wait()
  pltpu.make_async_copy(ei_hbm.at[pl.ds(base, EPW)], dst_v, sem_b).wait()
  plsc.subcore_barrier()

  def _gather(j, buf, sem):
    pltpu.async_copy(y_hbm.at[src_v.at[pl.ds(j * CH, CH)]], buf, sem)

  def _gwait(buf, sem):
    pltpu.make_async_copy(
        y_hbm.at[src_v.at[pl.ds(0, CH)]], buf, sem).wait()

  def _scat(j, buf):
    pltpu.sync_copy(buf, z_sh.at[dst_v.at[pl.ds(j * CH, CH)]], add=True)

  def _clamp(j):
    return pl.multiple_of(jnp.minimum(j * CH, (NCH - 1) * CH), 8)

  _gather(0, rows_a, sem_a)
  _gather(1, rows_b, sem_b)

  @pl.loop(0, NCH // 3)
  def _(g):
    j = g * 3
    _gather(j + 2, rows_c, sem_c)
    _gwait(rows_a, sem_a)
    _scat(j, rows_a)
    pltpu.async_copy(y_hbm.at[src_v.at[pl.ds(_clamp(j + 3), CH)]],
                     rows_a, sem_a)
    _gwait(rows_b, sem_b)
    _scat(j + 1, rows_b)
    pltpu.async_copy(y_hbm.at[src_v.at[pl.ds(_clamp(j + 4), CH)]],
                     rows_b, sem_b)
    _gwait(rows_c, sem_c)
    _scat(j + 2, rows_c)

  _gwait(rows_b, sem_b)
  pltpu.make_async_copy(
      y_hbm.at[src_v.at[pl.ds(0, CH)]], rows_a, sem_a).wait()
  pltpu.async_copy(
      y_hbm.at[src_v.at[pl.ds(NCH * CH, CHT)]],
      rows_a.at[pl.ds(0, CHT)], sem_a).wait()
  pltpu.sync_copy(rows_a.at[pl.ds(0, CHT)],
                  z_sh.at[dst_v.at[pl.ds(NCH * CH, CHT)]], add=True)

  plsc.subcore_barrier()
  pltpu.sync_copy(z_sh.at[pl.ds(row0, RPS)],
                  out_hbm.at[c, pl.ds(row0, RPS)])


def _mm_body(x_ref, w_ref, deg_ref, y_ref, dinv_ref):
  deg = deg_ref[0, pl.ds(0, N), 0] + deg_ref[1, pl.ds(0, N), 0] + 1.0
  dinv = lax.rsqrt(deg)
  xw = jnp.dot(x_ref[...], w_ref[...], preferred_element_type=jnp.float32)
  y_ref[...] = xw * dinv[:, None]
  dinv_ref[...] = dinv[:, None]


def _mm_call(x, w, deg_parts):
  return pl.pallas_call(
      _mm_body,
      out_shape=(jax.ShapeDtypeStruct((N, H), jnp.float32),
                 jax.ShapeDtypeStruct((N, 1), jnp.float32)),
  )(x, w, deg_parts)


def _head_body(z_ref, y_ref, dinv_ref, bg_ref, w1_ref, b1_ref, w2_ref, b2_ref,
               w3_ref, b3_ref, o_ref):
  z = z_ref[0, pl.ds(0, N), :] + z_ref[1, pl.ds(0, N), :] + y_ref[...]
  h = jax.nn.relu(z * dinv_ref[...] + bg_ref[...])
  h = jax.nn.relu(
      jnp.dot(h, w1_ref[...], preferred_element_type=jnp.float32) + b1_ref[...])
  h = jax.nn.relu(
      jnp.dot(h, w2_ref[...], preferred_element_type=jnp.float32) + b2_ref[...])
  h = jnp.dot(h, w3_ref[...], preferred_element_type=jnp.float32) + b3_ref[...]
  m = jnp.max(h, axis=1, keepdims=True)
  lse = jnp.log(jnp.sum(jnp.exp(h - m), axis=1, keepdims=True))
  o_ref[...] = h - m - lse


def _head_call(z_parts, y, dinv, bg, w1, b1, w2, b2, w3, b3):
  return pl.pallas_call(
      _head_body,
      out_shape=jax.ShapeDtypeStruct((N, C), jnp.float32),
  )(z_parts, y, dinv, bg, w1, b1, w2, b2, w3, b3)


def kernel(x, edge_index, W_gcn, b_gcn, W1, b1, W2, b2, W3, b3):
  ei = edge_index.reshape(-1)
  deg_parts = _deg_kernel(ei)
  y, dinv = _mm_call(x, W_gcn, deg_parts)
  z_parts = _msg_kernel(y, ei)
  return _head_call(z_parts, y, dinv,
                    b_gcn.reshape(1, H), W1, b1.reshape(1, 32),
                    W2, b2.reshape(1, 16), W3, b3.reshape(1, C))
